# baseline (speedup 1.0000x reference)
import jax
import jax.numpy as jnp
from jax import lax
from jax.experimental import pallas as pl
from jax.experimental.pallas import tpu as pltpu
from jax.experimental.pallas import tpu_sc as plsc

M = 64
N = 4096
B = 2
F = 2048

NWORK = 32
NCOL = N // NWORK
RCH = 8


def _gate_kernel(u_ref, w_ref, l_ref, el_ref, loss_ref, srep_ref):
    logits = l_ref[...]
    p = jax.nn.sigmoid(logits)
    b = (u_ref[...] < p).astype(jnp.float32)
    scale = jnp.sum(w_ref[...] * b, axis=0)
    log_prob = b * jax.nn.log_sigmoid(logits) + (1.0 - b) * jax.nn.log_sigmoid(-logits)
    loss_ref[...] = el_ref[...] + jnp.sum(log_prob, axis=0, keepdims=True)
    srep_ref[...] = jnp.broadcast_to(scale[:, None], (N, 16))


NCH = B * (NCOL // RCH)


def _sc_combine(x_hbm, srep_hbm, out_hbm, srep_v, xb0, xb1, ob0, ob1,
                sem_srep, si0, si1, so0, so1):
    wid = lax.axis_index("s") * 2 + lax.axis_index("c")
    n0 = wid * NCOL
    pltpu.async_copy(srep_hbm.at[pl.ds(n0, NCOL), :], srep_v, sem_srep)

    def src_of(g):
        bb = g // (NCOL // RCH)
        cc = g % (NCOL // RCH)
        row0 = bb * N + n0 + cc * RCH
        return x_hbm.at[pl.ds(row0, RCH), :]

    def dst_of(g):
        bb = g // (NCOL // RCH)
        cc = g % (NCOL // RCH)
        row0 = bb * N + n0 + cc * RCH
        return out_hbm.at[pl.ds(row0, RCH), :]

    pltpu.async_copy(src_of(0), xb0, si0)
    pltpu.async_copy(src_of(1), xb1, si1)
    pltpu.make_async_copy(srep_hbm.at[pl.ds(n0, NCOL), :], srep_v, sem_srep).wait()

    def compute(xbuf, obuf, g):
        cc = g % (NCOL // RCH)
        svs = [srep_v[cc * RCH + j, :] for j in range(RCH)]

        @plsc.parallel_loop(0, F // 16, 1, unroll=2)
        def col_body(c):
            for j in range(RCH):
                obuf[j, pl.ds(c * 16, 16)] = xbuf[j, pl.ds(c * 16, 16)] * svs[j]

    def step(s, _):
        for (xb, ob, si, so, off) in ((xb0, ob0, si0, so0, 0),
                                      (xb1, ob1, si1, so1, 1)):
            g = 2 * s + off
            pltpu.make_async_copy(src_of(g), xb, si).wait()

            @pl.when(s > 0)
            def _():
                pltpu.make_async_copy(ob, dst_of(g - 2), so).wait()

            compute(xb, ob, g)

            @pl.when(s < NCH // 2 - 1)
            def _():
                pltpu.async_copy(src_of(g + 2), xb, si)

            pltpu.async_copy(ob, dst_of(g), so)
        return 0

    lax.fori_loop(0, NCH // 2, step, 0)
    pltpu.make_async_copy(ob0, dst_of(NCH - 2), so0).wait()
    pltpu.make_async_copy(ob1, dst_of(NCH - 1), so1).wait()


def kernel(x, extra_loss, weights, logits):
    u = jax.random.uniform(jax.random.key(42), (M, N), jnp.float32)
    el2d = extra_loss.reshape(1, N)

    loss, srep = pl.pallas_call(
        _gate_kernel,
        out_shape=[
            jax.ShapeDtypeStruct((1, N), jnp.float32),
            jax.ShapeDtypeStruct((N, 16), jnp.float32),
        ],
    )(u, weights, logits, el2d)

    x2 = x.reshape(B * N, F)
    mesh = plsc.VectorSubcoreMesh(core_axis_name="c", subcore_axis_name="s")
    out2 = pl.kernel(
        _sc_combine,
        out_type=jax.ShapeDtypeStruct((B * N, F), jnp.float32),
        mesh=mesh,
        scratch_types=[
            pltpu.VMEM((NCOL, 16), jnp.float32),
            pltpu.VMEM((RCH, F), jnp.float32),
            pltpu.VMEM((RCH, F), jnp.float32),
            pltpu.VMEM((RCH, F), jnp.float32),
            pltpu.VMEM((RCH, F), jnp.float32),
            pltpu.SemaphoreType.DMA,
            pltpu.SemaphoreType.DMA,
            pltpu.SemaphoreType.DMA,
            pltpu.SemaphoreType.DMA,
            pltpu.SemaphoreType.DMA,
        ],
    )(x2, srep)

    return out2.reshape(B, N, F), loss.reshape(N)


# speedup vs baseline: 1.7203x; 1.7203x over previous
import jax
import jax.numpy as jnp
import numpy as np
from jax import lax
from jax.experimental import pallas as pl
from jax.experimental.pallas import tpu as pltpu

M = 64
N = 4096
B = 2
F = 2048

NBLK = 1024

_KS = (0, 42, 0 ^ 42 ^ 0x1BD11BDA)
_ROT = ((13, 15, 26, 6), (17, 29, 16, 24))


def _rotl(x, r):
    return (x << np.uint32(r)) | (x >> np.uint32(32 - r))


def _threefry_bits(flat):
    ks = tuple(np.uint32(k) for k in _KS)
    x0 = jnp.zeros_like(flat) + ks[0]
    x1 = flat + ks[1]
    for r in range(5):
        for rr in _ROT[r % 2]:
            x0 = x0 + x1
            x1 = _rotl(x1, rr) ^ x0
        x0 = x0 + ks[(r + 1) % 3]
        x1 = x1 + ks[(r + 2) % 3] + np.uint32(r + 1)
    return x0 ^ x1


def _gating_kernel(w_ref, l_ref, el_ref, x_ref, out_ref, loss_ref, scale_ref):
    i = pl.program_id(0)
    b = pl.program_id(1)

    @pl.when(b == 0)
    def _():
        row = lax.broadcasted_iota(np.uint32, (M, NBLK), 0)
        col = lax.broadcasted_iota(np.uint32, (M, NBLK), 1)
        flat = row * np.uint32(N) + col + i.astype(np.uint32) * np.uint32(NBLK)
        bits = _threefry_bits(flat)
        fbits = (bits >> np.uint32(9)) | np.uint32(0x3F800000)
        u = lax.bitcast_convert_type(fbits, jnp.float32) - 1.0

        logits = l_ref[...]
        p = jax.nn.sigmoid(logits)
        bern = (u < p).astype(jnp.float32)
        scale_ref[...] = jnp.sum(w_ref[...] * bern, axis=0, keepdims=True)
        log_prob = (bern * jax.nn.log_sigmoid(logits)
                    + (1.0 - bern) * jax.nn.log_sigmoid(-logits))
        loss_ref[...] = el_ref[...] + jnp.sum(log_prob, axis=0, keepdims=True)

    out_ref[...] = x_ref[...] * scale_ref[...].reshape(1, NBLK, 1)


def kernel(x, extra_loss, weights, logits):
    el2d = extra_loss.reshape(1, N)
    grid = (N // NBLK, B)
    out, loss = pl.pallas_call(
        _gating_kernel,
        grid=grid,
        in_specs=[
            pl.BlockSpec((M, NBLK), lambda i, b: (0, i)),
            pl.BlockSpec((M, NBLK), lambda i, b: (0, i)),
            pl.BlockSpec((1, NBLK), lambda i, b: (0, i)),
            pl.BlockSpec((1, NBLK, F), lambda i, b: (b, i, 0)),
        ],
        out_specs=[
            pl.BlockSpec((1, NBLK, F), lambda i, b: (b, i, 0)),
            pl.BlockSpec((1, NBLK), lambda i, b: (0, i)),
        ],
        out_shape=[
            jax.ShapeDtypeStruct((B, N, F), jnp.float32),
            jax.ShapeDtypeStruct((1, N), jnp.float32),
        ],
        scratch_shapes=[pltpu.VMEM((1, NBLK), jnp.float32)],
        compiler_params=pltpu.CompilerParams(
            dimension_semantics=("arbitrary", "arbitrary"),
        ),
    )(weights, logits, el2d, x)
    return out, loss.reshape(N)
